# Initial kernel scaffold; baseline (speedup 1.0000x reference)
import functools

import jax
import jax.numpy as jnp
from jax import lax
from jax.experimental import pallas as pl
from jax.experimental.pallas import tpu as pltpu
from jax.experimental.pallas import tpu_sc as plsc

N = 10000
NPAD = 10240
E = 320000
H = 128
DIN = 128
G = 256
NC = 2
NS = 16
NW = NC * NS
CH = 128
CPT = 80
EPT = CH * CPT
EPAD = EPT * NW
RPS = NPAD // NS


def _edge_layer(x, src2, dst2, ea2, we, be, zrows):
    mesh = plsc.VectorSubcoreMesh(
        core_axis_name="c", subcore_axis_name="s",
        num_cores=NC, num_subcores=NS)

    @functools.partial(
        pl.kernel,
        out_type=jax.ShapeDtypeStruct((NC, NPAD, H), jnp.float32),
        mesh=mesh,
        scratch_types=[
            pltpu.VMEM((CPT, CH), jnp.int32),
            pltpu.VMEM((CPT, CH), jnp.int32),
            pltpu.VMEM((CPT, CH), jnp.float32),
            pltpu.VMEM((CH, H), jnp.float32),
            pltpu.VMEM((H,), jnp.float32),
            pltpu.VMEM((H,), jnp.float32),
            pltpu.VMEM_SHARED((NPAD, H), jnp.float32),
            pltpu.SemaphoreType.DMA,
        ],
    )
    def k(x_hbm, src_hbm, dst_hbm, ea_hbm, we_hbm, be_hbm, z_hbm, out_hbm,
          srcv, dstv, eav, buf, wev, bev, acc, sem):
        c = lax.axis_index("c")
        s = lax.axis_index("s")
        wid = c * NS + s
        cbase = wid * CPT
        pltpu.sync_copy(src_hbm.at[pl.ds(cbase, CPT)], srcv)
        pltpu.sync_copy(dst_hbm.at[pl.ds(cbase, CPT)], dstv)
        pltpu.sync_copy(ea_hbm.at[pl.ds(cbase, CPT)], eav)
        pltpu.sync_copy(we_hbm, wev)
        pltpu.sync_copy(be_hbm, bev)
        pltpu.sync_copy(z_hbm, acc.at[pl.ds(s * RPS, RPS)])
        plsc.subcore_barrier()

        wes = [wev[pl.ds(16 * j, 16)] for j in range(8)]
        bes = [bev[pl.ds(16 * j, 16)] for j in range(8)]

        def chunk_body(t, carry):
            pltpu.async_copy(x_hbm.at[srcv.at[t]], buf, sem).wait()

            def grp(g, c2):
                av = eav[t, pl.ds(g * 16, 16)]
                for i in range(16):
                    ai = av[i]
                    r = g * 16 + i
                    for j in range(8):
                        col = pl.ds(j * 16, 16)
                        msg = jnp.maximum(
                            buf[r, col] + (ai * wes[j] + bes[j]), 0.0)
                        buf[r, col] = msg
                return c2

            lax.fori_loop(0, 8, grp, 0)
            pltpu.sync_copy(buf, acc.at[dstv.at[t]], add=True)
            return carry

        lax.fori_loop(0, CPT, chunk_body, 0)
        plsc.subcore_barrier()
        pltpu.sync_copy(acc.at[pl.ds(s * RPS, RPS)],
                        out_hbm.at[c, pl.ds(s * RPS, RPS)])

    return k(x, src2, dst2, ea2, we, be, zrows)


def _proj(nf, w, b):
    def body(x_ref, w_ref, b_ref, o_ref):
        o_ref[...] = jnp.dot(x_ref[...], w_ref[...],
                             preferred_element_type=jnp.float32) + b_ref[...]

    return pl.pallas_call(
        body,
        grid=(NPAD // 1024,),
        in_specs=[
            pl.BlockSpec((1024, DIN), lambda i: (i, 0)),
            pl.BlockSpec((DIN, H), lambda i: (0, 0)),
            pl.BlockSpec((1, H), lambda i: (0, 0)),
        ],
        out_specs=pl.BlockSpec((1024, H), lambda i: (i, 0)),
        out_shape=jax.ShapeDtypeStruct((NPAD, H), jnp.float32),
    )(nf, w, b)


def _mlp(x, parts, w1, b1, w2, b2):
    def body(x_ref, p_ref, w1_ref, b1_ref, w2_ref, b2_ref, o_ref):
        t = x_ref[...] + p_ref[0] + p_ref[1]
        h = jnp.maximum(jnp.dot(t, w1_ref[...],
                                preferred_element_type=jnp.float32)
                        + b1_ref[...], 0.0)
        h = jnp.dot(h, w2_ref[...],
                    preferred_element_type=jnp.float32) + b2_ref[...]
        o_ref[...] = jnp.maximum(h, 0.0)

    return pl.pallas_call(
        body,
        grid=(NPAD // 1024,),
        in_specs=[
            pl.BlockSpec((1024, H), lambda i: (i, 0)),
            pl.BlockSpec((NC, 1024, H), lambda i: (0, i, 0)),
            pl.BlockSpec((H, H), lambda i: (0, 0)),
            pl.BlockSpec((1, H), lambda i: (0, 0)),
            pl.BlockSpec((H, H), lambda i: (0, 0)),
            pl.BlockSpec((1, H), lambda i: (0, 0)),
        ],
        out_specs=pl.BlockSpec((1024, H), lambda i: (i, 0)),
        out_shape=jax.ShapeDtypeStruct((NPAD, H), jnp.float32),
    )(x, parts, w1, b1, w2, b2)


def _head(x3, mem, pw1, pb1, pw2, pb2, pw3, pb3):
    def body(x_ref, m_ref, w1_ref, b1_ref, w2_ref, b2_ref, w3_ref, b3_ref,
             o_ref):
        m = m_ref[...]
        gids = lax.broadcasted_iota(jnp.int32, (1, G), 1)
        onehot = (m == gids).astype(jnp.float32)
        sums = lax.dot_general(onehot, x_ref[...],
                               (((0,), (0,)), ((), ())),
                               preferred_element_type=jnp.float32)
        counts = jnp.sum(onehot, axis=0)
        z = sums / jnp.maximum(counts, 1.0)[:, None]
        ctx = jnp.broadcast_to(jnp.mean(z, axis=0, keepdims=True), z.shape)
        hcat = jnp.concatenate([z, ctx], axis=1)
        h = jnp.maximum(jnp.dot(hcat, w1_ref[...],
                                preferred_element_type=jnp.float32)
                        + b1_ref[...], 0.0)
        h = jnp.maximum(jnp.dot(h, w2_ref[...],
                                preferred_element_type=jnp.float32)
                        + b2_ref[...], 0.0)
        o_ref[...] = jnp.dot(h, w3_ref[...],
                             preferred_element_type=jnp.float32) + b3_ref[...]

    return pl.pallas_call(
        body,
        out_shape=jax.ShapeDtypeStruct((G, 1), jnp.float32),
    )(x3, mem, pw1, pb1, pw2, pb2, pw3, pb3)


def kernel(node_features, edge_index, edge_attr, membership, W_in, b_in,
           W_edge, b_edge, W1, b1, W2, b2, P1, p1, P2, p2, P3, p3):
    src = edge_index[0].astype(jnp.int32)
    dst = edge_index[1].astype(jnp.int32)
    ea = edge_attr[:, 0]
    pad = EPAD - E
    src2 = jnp.concatenate([src, jnp.zeros((pad,), jnp.int32)]).reshape(
        EPAD // CH, CH)
    dst2 = jnp.concatenate([dst, jnp.full((pad,), N, jnp.int32)]).reshape(
        EPAD // CH, CH)
    ea2 = jnp.concatenate([ea, jnp.zeros((pad,), jnp.float32)]).reshape(
        EPAD // CH, CH)
    nf = jnp.concatenate(
        [node_features, jnp.zeros((NPAD - N, DIN), jnp.float32)])
    mem = jnp.concatenate(
        [membership.astype(jnp.int32), jnp.full((NPAD - N,), G, jnp.int32)]
    ).reshape(NPAD, 1)
    we = W_edge[0]
    zrows = jnp.zeros((RPS, H), jnp.float32)

    x = _proj(nf, W_in, b_in.reshape(1, H))
    for l in range(3):
        parts = _edge_layer(x, src2, dst2, ea2, we, b_edge, zrows)
        x = _mlp(x, parts, W1[l], b1[l].reshape(1, H), W2[l],
                 b2[l].reshape(1, H))
    out = _head(x, mem, P1, p1.reshape(1, H), P2, p2.reshape(1, H), P3,
                p3.reshape(1, 1))
    return out[:, 0]


# speedup vs baseline: 1.7687x; 1.7687x over previous
import functools

import jax
import jax.numpy as jnp
from jax import lax
from jax.experimental import pallas as pl
from jax.experimental.pallas import tpu as pltpu
from jax.experimental.pallas import tpu_sc as plsc

N = 10000
NPAD = 10240
E = 320000
H = 128
DIN = 128
G = 256
NC = 2
NS = 16
NW = NC * NS
CH = 128
RT = NPAD // NW


def _edge_layer(x, srcs, dsts, eas, lo16, hi16, chlo16, nch16, we, be, zrows):
    mesh = plsc.VectorSubcoreMesh(
        core_axis_name="c", subcore_axis_name="s",
        num_cores=NC, num_subcores=NS)

    @functools.partial(
        pl.kernel,
        out_type=jax.ShapeDtypeStruct((NPAD, H), jnp.float32),
        mesh=mesh,
        scratch_types=[
            pltpu.VMEM((CH,), jnp.int32),
            pltpu.VMEM((CH,), jnp.int32),
            pltpu.VMEM((CH,), jnp.float32),
            pltpu.VMEM((CH, H), jnp.float32),
            pltpu.VMEM((16,), jnp.int32),
            pltpu.VMEM((16,), jnp.int32),
            pltpu.VMEM((16,), jnp.int32),
            pltpu.VMEM((16,), jnp.int32),
            pltpu.VMEM((H,), jnp.float32),
            pltpu.VMEM((H,), jnp.float32),
            pltpu.VMEM((RT, H), jnp.float32),
            pltpu.SemaphoreType.DMA,
        ],
    )
    def k(x_hbm, src_hbm, dst_hbm, ea_hbm, lo_hbm, hi_hbm, chlo_hbm,
          nch_hbm, we_hbm, be_hbm, z_hbm, out_hbm,
          srcv, dstv, eav, buf, lov, hiv, chlov, nchv, wev, bev, acc, sem):
        c = lax.axis_index("c")
        s = lax.axis_index("s")
        w = c * NS + s
        base_row = w * RT
        pltpu.sync_copy(lo_hbm.at[w], lov)
        pltpu.sync_copy(hi_hbm.at[w], hiv)
        pltpu.sync_copy(chlo_hbm.at[w], chlov)
        pltpu.sync_copy(nch_hbm.at[w], nchv)
        pltpu.sync_copy(we_hbm, wev)
        pltpu.sync_copy(be_hbm, bev)
        pltpu.sync_copy(z_hbm, acc)

        lo_vec = lov[...]
        hi_vec = hiv[...]
        chlo = chlov[...][0]
        nch = nchv[...][0]
        wes = [wev[pl.ds(16 * j, 16)] for j in range(8)]
        bes = [bev[pl.ds(16 * j, 16)] for j in range(8)]
        iota16 = lax.iota(jnp.int32, 16)

        def chunk_body(t, carry):
            off = (chlo + t) * CH
            pltpu.sync_copy(src_hbm.at[pl.ds(off, CH)], srcv)
            pltpu.sync_copy(dst_hbm.at[pl.ds(off, CH)], dstv)
            pltpu.sync_copy(ea_hbm.at[pl.ds(off, CH)], eav)
            pltpu.async_copy(x_hbm.at[srcv], buf, sem).wait()

            def grp(g, c2):
                av = eav[pl.ds(g * 16, 16)]
                dvv = dstv[pl.ds(g * 16, 16)]
                pos = off + g * 16 + iota16
                inr = (pos >= lo_vec) & (pos < hi_vec)
                mf = jnp.where(inr, jnp.float32(1.0), jnp.float32(0.0))
                for i in range(16):
                    ai = av[i]
                    fi = mf[i]
                    d = jnp.clip(dvv[i] - base_row, 0, RT - 1)
                    r = g * 16 + i
                    for j in range(8):
                        col = pl.ds(j * 16, 16)
                        msg = jnp.maximum(
                            buf[r, col] + (ai * wes[j] + bes[j]), 0.0) * fi
                        plsc.addupdate(acc.at[d, col], msg)
                return c2

            lax.fori_loop(0, 8, grp, 0)
            return carry

        lax.fori_loop(0, nch, chunk_body, 0)
        pltpu.sync_copy(acc, out_hbm.at[pl.ds(base_row, RT)])

    return k(x, srcs, dsts, eas, lo16, hi16, chlo16, nch16, we, be, zrows)


def _proj(nf, w, b):
    def body(x_ref, w_ref, b_ref, o_ref):
        o_ref[...] = jnp.dot(x_ref[...], w_ref[...],
                             preferred_element_type=jnp.float32) + b_ref[...]

    return pl.pallas_call(
        body,
        grid=(NPAD // 1024,),
        in_specs=[
            pl.BlockSpec((1024, DIN), lambda i: (i, 0)),
            pl.BlockSpec((DIN, H), lambda i: (0, 0)),
            pl.BlockSpec((1, H), lambda i: (0, 0)),
        ],
        out_specs=pl.BlockSpec((1024, H), lambda i: (i, 0)),
        out_shape=jax.ShapeDtypeStruct((NPAD, H), jnp.float32),
    )(nf, w, b)


def _mlp(x, aggr, w1, b1, w2, b2):
    def body(x_ref, a_ref, w1_ref, b1_ref, w2_ref, b2_ref, o_ref):
        t = x_ref[...] + a_ref[...]
        h = jnp.maximum(jnp.dot(t, w1_ref[...],
                                preferred_element_type=jnp.float32)
                        + b1_ref[...], 0.0)
        h = jnp.dot(h, w2_ref[...],
                    preferred_element_type=jnp.float32) + b2_ref[...]
        o_ref[...] = jnp.maximum(h, 0.0)

    return pl.pallas_call(
        body,
        grid=(NPAD // 1024,),
        in_specs=[
            pl.BlockSpec((1024, H), lambda i: (i, 0)),
            pl.BlockSpec((1024, H), lambda i: (i, 0)),
            pl.BlockSpec((H, H), lambda i: (0, 0)),
            pl.BlockSpec((1, H), lambda i: (0, 0)),
            pl.BlockSpec((H, H), lambda i: (0, 0)),
            pl.BlockSpec((1, H), lambda i: (0, 0)),
        ],
        out_specs=pl.BlockSpec((1024, H), lambda i: (i, 0)),
        out_shape=jax.ShapeDtypeStruct((NPAD, H), jnp.float32),
    )(x, aggr, w1, b1, w2, b2)


def _head(x3, mem, pw1, pb1, pw2, pb2, pw3, pb3):
    def body(x_ref, m_ref, w1_ref, b1_ref, w2_ref, b2_ref, w3_ref, b3_ref,
             o_ref):
        m = m_ref[...]
        gids = lax.broadcasted_iota(jnp.int32, (1, G), 1)
        onehot = (m == gids).astype(jnp.float32)
        sums = lax.dot_general(onehot, x_ref[...],
                               (((0,), (0,)), ((), ())),
                               preferred_element_type=jnp.float32)
        counts = jnp.sum(onehot, axis=0)
        z = sums / jnp.maximum(counts, 1.0)[:, None]
        ctx = jnp.broadcast_to(jnp.mean(z, axis=0, keepdims=True), z.shape)
        hcat = jnp.concatenate([z, ctx], axis=1)
        h = jnp.maximum(jnp.dot(hcat, w1_ref[...],
                                preferred_element_type=jnp.float32)
                        + b1_ref[...], 0.0)
        h = jnp.maximum(jnp.dot(h, w2_ref[...],
                                preferred_element_type=jnp.float32)
                        + b2_ref[...], 0.0)
        o_ref[...] = jnp.dot(h, w3_ref[...],
                             preferred_element_type=jnp.float32) + b3_ref[...]

    return pl.pallas_call(
        body,
        out_shape=jax.ShapeDtypeStruct((G, 1), jnp.float32),
    )(x3, mem, pw1, pb1, pw2, pb2, pw3, pb3)


def kernel(node_features, edge_index, edge_attr, membership, W_in, b_in,
           W_edge, b_edge, W1, b1, W2, b2, P1, p1, P2, p2, P3, p3):
    src = edge_index[0].astype(jnp.int32)
    dst = edge_index[1].astype(jnp.int32)
    ea = edge_attr[:, 0]
    order = jnp.argsort(dst, stable=True)
    srcs = src[order]
    dsts = dst[order]
    eas = ea[order]
    lo = jnp.searchsorted(dsts, jnp.arange(NW, dtype=jnp.int32) * RT,
                          side='left').astype(jnp.int32)
    hi = jnp.concatenate([lo[1:], jnp.full((1,), E, jnp.int32)])
    chlo = lo // CH
    nch = jnp.maximum((hi + CH - 1) // CH - chlo, 0).astype(jnp.int32)
    rep = lambda v: jnp.broadcast_to(v[:, None], (NW, 16))
    lo16, hi16, chlo16, nch16 = rep(lo), rep(hi), rep(chlo), rep(nch)

    nf = jnp.concatenate(
        [node_features, jnp.zeros((NPAD - N, DIN), jnp.float32)])
    mem = jnp.concatenate(
        [membership.astype(jnp.int32), jnp.full((NPAD - N,), G, jnp.int32)]
    ).reshape(NPAD, 1)
    we = W_edge[0]
    zrows = jnp.zeros((RT, H), jnp.float32)

    x = _proj(nf, W_in, b_in.reshape(1, H))
    for l in range(3):
        aggr = _edge_layer(x, srcs, dsts, eas, lo16, hi16, chlo16, nch16,
                           we, b_edge, zrows)
        x = _mlp(x, aggr, W1[l], b1[l].reshape(1, H), W2[l],
                 b2[l].reshape(1, H))
    out = _head(x, mem, P1, p1.reshape(1, H), P2, p2.reshape(1, H), P3,
                p3.reshape(1, 1))
    return out[:, 0]
